# baseline (speedup 1.0000x reference)
import functools

import jax
import jax.numpy as jnp
from jax import lax
from jax.experimental import pallas as pl
from jax.experimental.pallas import tpu as pltpu
from jax.experimental.pallas import tpu_sc as plsc

N_E = 1024
E_DIM = 64
BETA = 0.25
B = 32
T = 16
BT = B * T
NCOL = N_E + 1
NPAD = 1040
BIG = 1e30


def _tc_dist(z_ref, w_ref, d_ref, w128_ref):
    z3 = z_ref[...]
    z2d = z3.reshape(BT, E_DIM)
    w = w_ref[...]
    wpad = jnp.concatenate(
        [w, jnp.zeros((NPAD - NCOL, E_DIM), jnp.float32)], axis=0)
    z2 = jnp.sum(z2d * z2d, axis=1, keepdims=True)
    ww = wpad * wpad
    w2r = lax.dot_general(jnp.ones((1, E_DIM), jnp.float32), ww,
                          (((1,), (1,)), ((), ())),
                          preferred_element_type=jnp.float32)
    colr = lax.broadcasted_iota(jnp.int32, (1, NPAD), 1)
    w2r = jnp.where(colr < NCOL, w2r, BIG)
    zw = lax.dot_general(z2d, wpad, (((1,), (1,)), ((), ())),
                         preferred_element_type=jnp.float32)
    d = (z2 + w2r) - 2.0 * zw
    d_ref[...] = d.reshape(B, T, NPAD)
    w128_ref[...] = jnp.concatenate(
        [w, jnp.zeros((NCOL, 128 - E_DIM), jnp.float32)], axis=1)


_sc_mesh = plsc.VectorSubcoreMesh(core_axis_name="c", subcore_axis_name="s")


@functools.partial(
    pl.kernel,
    mesh=_sc_mesh,
    compiler_params=pltpu.CompilerParams(needs_layout_passes=False),
    out_type=(
        jax.ShapeDtypeStruct((B, T, E_DIM), jnp.float32),
        jax.ShapeDtypeStruct((B, T), jnp.int32),
        jax.ShapeDtypeStruct((B, T, 16), jnp.float32),
    ),
    scratch_types=[
        pltpu.VMEM((T, NPAD), jnp.float32),
        pltpu.VMEM((T, E_DIM), jnp.float32),
        pltpu.VMEM((T, 128), jnp.float32),
        pltpu.VMEM((16,), jnp.int32),
        pltpu.VMEM((T, 16), jnp.float32),
        pltpu.VMEM((16,), jnp.float32),
        pltpu.VMEM((16,), jnp.int32),
        pltpu.VMEM((16,), jnp.float32),
        pltpu.SemaphoreType.DMA,
    ],
)
def _sc_scan(d_hbm, z_hbm, w_hbm, zq_hbm, ind_hbm, dsel_hbm,
             dv, zv, wrows, idxv, dsp, bval, bidx, dselr, sem):
    wid = lax.axis_index("s") * 2 + lax.axis_index("c")
    pltpu.sync_copy(d_hbm.at[wid], dv)
    pltpu.sync_copy(z_hbm.at[wid], zv)
    iota = lax.broadcasted_iota(jnp.int32, (16,), 0)
    zero16 = jnp.zeros((16,), jnp.int32)

    def amin_body(c, carry):
        best, besti = carry
        vv = dv[0, pl.ds(c * 16, 16)]
        ci = iota + c * 16
        take = vv < best
        return (jnp.where(take, vv, best), jnp.where(take, ci, besti))

    best, besti = lax.fori_loop(1, NPAD // 16, amin_body,
                                (dv[0, pl.ds(0, 16)], iota))
    for k in (1, 2, 4, 8):
        bval[...] = best
        bidx[...] = besti
        partner = iota ^ k
        gv = plsc.load_gather(bval, [partner])
        gi = plsc.load_gather(bidx, [partner])
        take = (gv < best) | ((gv == best) & (gi < besti))
        best = jnp.where(take, gv, best)
        besti = jnp.where(take, gi, besti)
    indv = jnp.minimum(besti, N_E - 1)

    dselv = plsc.load_gather(dv, [zero16, indv])
    dsel_row = jnp.where(iota == 0, dselv, 0.0)
    ind_row = jnp.where(iota == 0, indv, 0)
    ones16 = jnp.full((16,), 1, jnp.int32)

    def scan_body(t, carry):
        indv, ind_row, dsel_row, tv = carry
        indnv = jnp.minimum(indv + 1, N_E - 1)
        gh = plsc.load_gather(dv, [tv, indv])
        gn = plsc.load_gather(dv, [tv, indnv])
        keep = gh <= gn
        indv = jnp.where(keep, indv, indnv)
        dsel = jnp.where(keep, gh, gn)
        mask = iota == tv
        ind_row = jnp.where(mask, indv, ind_row)
        dsel_row = jnp.where(mask, dsel, dsel_row)
        return (indv, ind_row, dsel_row, tv + ones16)

    indv, ind_row, dsel_row, _ = lax.fori_loop(
        1, T, scan_body, (indv, ind_row, dsel_row, ones16))

    idxv[...] = ind_row
    dselr[...] = dsel_row
    pltpu.sync_copy(idxv, ind_hbm.at[wid])
    for tt in range(T):
        dsp[tt, pl.ds(0, 16)] = plsc.load_gather(
            dselr, [jnp.full((16,), tt, jnp.int32)])
    pltpu.sync_copy(dsp, dsel_hbm.at[wid])

    pltpu.async_copy(w_hbm.at[idxv], wrows, sem).wait()
    for tt in range(T):
        for e in range(E_DIM // 16):
            zz = zv[tt, pl.ds(e * 16, 16)]
            wv = wrows[tt, pl.ds(e * 16, 16)]
            zv[tt, pl.ds(e * 16, 16)] = zz + (wv - zz)
    pltpu.sync_copy(zv, zq_hbm.at[wid])


def _tc_loss(d_ref, dsel_ref, ind_ref, loss_ref, v_ref):
    d = d_ref[...].reshape(BT, NPAD)
    dsel = dsel_ref[...].reshape(BT, 16)[:, 0:1]
    eps = 1e-06 / N_E
    terms = jnp.maximum((dsel - d) + eps, 0.0)
    lc = jnp.sum(jnp.sum(terms, axis=1, keepdims=True), axis=0,
                 keepdims=True) / float(BT * NCOL)
    loss_ref[...] = BETA * lc + lc
    ind = ind_ref[...]
    rng = (jnp.max(ind, axis=1, keepdims=True)
           - jnp.min(ind, axis=1, keepdims=True))
    v_ref[...] = jnp.max(rng, axis=0, keepdims=True)


def kernel(z, W):
    d, w128 = pl.pallas_call(
        _tc_dist,
        out_shape=[
            jax.ShapeDtypeStruct((B, T, NPAD), jnp.float32),
            jax.ShapeDtypeStruct((NCOL, 128), jnp.float32),
        ],
    )(z, W)
    zq, ind, dsel = _sc_scan(d, z, w128)
    loss, v = pl.pallas_call(
        _tc_loss,
        out_shape=[
            jax.ShapeDtypeStruct((1, 1), jnp.float32),
            jax.ShapeDtypeStruct((1, 1), jnp.int32),
        ],
    )(d, dsel, ind)
    return (zq, loss.reshape(()), ind, v.reshape(()))


# speedup vs baseline: 1.0066x; 1.0066x over previous
import functools

import jax
import jax.numpy as jnp
from jax import lax
from jax.experimental import pallas as pl
from jax.experimental.pallas import tpu as pltpu
from jax.experimental.pallas import tpu_sc as plsc

N_E = 1024
E_DIM = 64
BETA = 0.25
B = 32
T = 16
BT = B * T
NCOL = N_E + 1
NPAD = 1040
BIG = 1e30


def _tc_dist(z_ref, w_ref, d_ref, w128_ref):
    z3 = z_ref[...]
    z2d = z3.reshape(BT, E_DIM)
    w = w_ref[...]
    wpad = jnp.concatenate(
        [w, jnp.zeros((NPAD - NCOL, E_DIM), jnp.float32)], axis=0)
    z2 = jnp.sum(z2d * z2d, axis=1, keepdims=True)
    ww = wpad * wpad
    w2r = lax.dot_general(jnp.ones((1, E_DIM), jnp.float32), ww,
                          (((1,), (1,)), ((), ())),
                          preferred_element_type=jnp.float32)
    colr = lax.broadcasted_iota(jnp.int32, (1, NPAD), 1)
    w2r = jnp.where(colr < NCOL, w2r, BIG)
    zw = lax.dot_general(z2d, wpad, (((1,), (1,)), ((), ())),
                         preferred_element_type=jnp.float32)
    d = (z2 + w2r) - 2.0 * zw
    d_ref[...] = d.reshape(B, T, NPAD)
    w128_ref[...] = jnp.concatenate(
        [w, jnp.zeros((NCOL, 128 - E_DIM), jnp.float32)], axis=1)


_sc_mesh = plsc.VectorSubcoreMesh(core_axis_name="c", subcore_axis_name="s")


@functools.partial(
    pl.kernel,
    mesh=_sc_mesh,
    compiler_params=pltpu.CompilerParams(needs_layout_passes=False),
    out_type=(
        jax.ShapeDtypeStruct((B, T, E_DIM), jnp.float32),
        jax.ShapeDtypeStruct((B, T), jnp.int32),
        jax.ShapeDtypeStruct((B, T, 16), jnp.float32),
    ),
    scratch_types=[
        pltpu.VMEM((T, NPAD), jnp.float32),
        pltpu.VMEM((T, E_DIM), jnp.float32),
        pltpu.VMEM((T, 128), jnp.float32),
        pltpu.VMEM((16,), jnp.int32),
        pltpu.VMEM((T, 16), jnp.float32),
        pltpu.VMEM((16,), jnp.float32),
        pltpu.VMEM((16,), jnp.int32),
        pltpu.VMEM((16,), jnp.float32),
        pltpu.SemaphoreType.DMA,
        pltpu.SemaphoreType.DMA,
        pltpu.SemaphoreType.DMA,
    ],
)
def _sc_scan(d_hbm, z_hbm, w_hbm, zq_hbm, ind_hbm, dsel_hbm,
             dv, zv, wrows, idxv, dsp, bval, bidx, dselr, sem, sem2, sem3):
    wid = lax.axis_index("s") * 2 + lax.axis_index("c")
    cp_d = pltpu.async_copy(d_hbm.at[wid], dv, sem)
    cp_z = pltpu.async_copy(z_hbm.at[wid], zv, sem2)
    cp_d.wait()
    iota = lax.broadcasted_iota(jnp.int32, (16,), 0)
    zero16 = jnp.zeros((16,), jnp.int32)

    best = dv[0, pl.ds(0, 16)]
    besti = iota
    for c in range(1, NPAD // 16):
        vv = dv[0, pl.ds(c * 16, 16)]
        ci = iota + c * 16
        take = vv < best
        best = jnp.where(take, vv, best)
        besti = jnp.where(take, ci, besti)
    for k in (1, 2, 4, 8):
        bval[...] = best
        bidx[...] = besti
        partner = iota ^ k
        gv = plsc.load_gather(bval, [partner])
        gi = plsc.load_gather(bidx, [partner])
        take = (gv < best) | ((gv == best) & (gi < besti))
        best = jnp.where(take, gv, best)
        besti = jnp.where(take, gi, besti)
    indv = jnp.minimum(besti, N_E - 1)

    dselv = plsc.load_gather(dv, [zero16, indv])
    dsel_row = jnp.where(iota == 0, dselv, 0.0)
    ind_row = jnp.where(iota == 0, indv, 0)
    for t in range(1, T):
        tv = jnp.full((16,), t, jnp.int32)
        indnv = jnp.minimum(indv + 1, N_E - 1)
        gh = plsc.load_gather(dv, [tv, indv])
        gn = plsc.load_gather(dv, [tv, indnv])
        keep = gh <= gn
        indv = jnp.where(keep, indv, indnv)
        dsel = jnp.where(keep, gh, gn)
        mask = iota == t
        ind_row = jnp.where(mask, indv, ind_row)
        dsel_row = jnp.where(mask, dsel, dsel_row)

    idxv[...] = ind_row
    dselr[...] = dsel_row
    pltpu.sync_copy(idxv, ind_hbm.at[wid])
    for tt in range(T):
        dsp[tt, pl.ds(0, 16)] = plsc.load_gather(
            dselr, [jnp.full((16,), tt, jnp.int32)])
    pltpu.sync_copy(dsp, dsel_hbm.at[wid])

    pltpu.async_copy(w_hbm.at[idxv], wrows, sem3).wait()
    cp_z.wait()
    for tt in range(T):
        for e in range(E_DIM // 16):
            zz = zv[tt, pl.ds(e * 16, 16)]
            wv = wrows[tt, pl.ds(e * 16, 16)]
            zv[tt, pl.ds(e * 16, 16)] = zz + (wv - zz)
    pltpu.sync_copy(zv, zq_hbm.at[wid])


def _tc_loss(d_ref, dsel_ref, ind_ref, loss_ref, v_ref):
    d = d_ref[...].reshape(BT, NPAD)
    dsel = dsel_ref[...].reshape(BT, 16)[:, 0:1]
    eps = 1e-06 / N_E
    terms = jnp.maximum((dsel - d) + eps, 0.0)
    lc = jnp.sum(jnp.sum(terms, axis=1, keepdims=True), axis=0,
                 keepdims=True) / float(BT * NCOL)
    loss_ref[...] = BETA * lc + lc
    ind = ind_ref[...]
    rng = (jnp.max(ind, axis=1, keepdims=True)
           - jnp.min(ind, axis=1, keepdims=True))
    v_ref[...] = jnp.max(rng, axis=0, keepdims=True)


def kernel(z, W):
    d, w128 = pl.pallas_call(
        _tc_dist,
        out_shape=[
            jax.ShapeDtypeStruct((B, T, NPAD), jnp.float32),
            jax.ShapeDtypeStruct((NCOL, 128), jnp.float32),
        ],
    )(z, W)
    zq, ind, dsel = _sc_scan(d, z, w128)
    loss, v = pl.pallas_call(
        _tc_loss,
        out_shape=[
            jax.ShapeDtypeStruct((1, 1), jnp.float32),
            jax.ShapeDtypeStruct((1, 1), jnp.int32),
        ],
    )(d, dsel, ind)
    return (zq, loss.reshape(()), ind, v.reshape(()))


# speedup vs baseline: 3.5189x; 3.4958x over previous
import jax
import jax.numpy as jnp
from jax import lax
from jax.experimental import pallas as pl

N_E = 1024
E_DIM = 64
BETA = 0.25
B = 32
T = 16
BT = B * T
NCOL = N_E + 1


def _vq_kernel(z_ref, w_ref, zq_ref, loss_ref, ind_ref, v_ref):
    z3 = z_ref[...]
    zt3 = jnp.swapaxes(z3, 0, 1)
    z2d = zt3.reshape(BT, E_DIM)
    w = w_ref[...]

    z2 = jnp.sum(z2d * z2d, axis=1, keepdims=True)
    ww = w * w
    w2r = lax.dot_general(
        jnp.ones((1, E_DIM), jnp.float32), ww,
        (((1,), (1,)), ((), ())), preferred_element_type=jnp.float32)
    zw = lax.dot_general(
        z2d, w, (((1,), (1,)), ((), ())),
        preferred_element_type=jnp.float32)
    d = (z2 + w2r) - 2.0 * zw

    col = lax.broadcasted_iota(jnp.int32, (B, NCOL), 1)
    eps = 1e-06 / N_E

    d0 = d[0:B, :]
    mn = jnp.min(d0, axis=1, keepdims=True)
    ind = jnp.min(jnp.where(d0 == mn, col, NCOL), axis=1, keepdims=True)
    ind = jnp.minimum(ind, N_E - 1)
    oh0 = col == ind
    dsel = jnp.sum(jnp.where(oh0, d0, 0.0), axis=1, keepdims=True)
    wsel = lax.dot_general(
        jnp.where(oh0, 1.0, 0.0), w, (((1,), (0,)), ((), ())),
        preferred_element_type=jnp.float32)
    lacc = jnp.maximum((dsel - d0) + eps, 0.0)

    minv = ind
    maxv = ind
    ind_cols = [ind]
    wsel_rows = [wsel]
    for t in range(1, T):
        dt = d[t * B:(t + 1) * B, :]
        indn = jnp.minimum(ind + 1, N_E - 1)
        ohh = col == ind
        ohn = col == indn
        here = jnp.sum(jnp.where(ohh, dt, 0.0), axis=1, keepdims=True)
        nxt = jnp.sum(jnp.where(ohn, dt, 0.0), axis=1, keepdims=True)
        keep = here <= nxt
        ind = jnp.where(keep, ind, indn)
        dsel = jnp.where(keep, here, nxt)
        wh = lax.dot_general(
            jnp.where(ohh, 1.0, 0.0), w, (((1,), (0,)), ((), ())),
            preferred_element_type=jnp.float32)
        wn = lax.dot_general(
            jnp.where(ohn, 1.0, 0.0), w, (((1,), (0,)), ((), ())),
            preferred_element_type=jnp.float32)
        wsel_rows.append(jnp.where(keep, wh, wn))
        lacc = lacc + jnp.maximum((dsel - dt) + eps, 0.0)
        ind_cols.append(ind)
        minv = jnp.minimum(minv, ind)
        maxv = jnp.maximum(maxv, ind)

    ind_ref[...] = jnp.concatenate(ind_cols, axis=1)
    lc = jnp.sum(jnp.sum(lacc, axis=1, keepdims=True), axis=0,
                 keepdims=True) / float(BT * NCOL)
    loss_ref[...] = BETA * lc + lc
    v_ref[...] = jnp.max(maxv - minv, axis=0, keepdims=True)

    wsel_t = jnp.concatenate(wsel_rows, axis=0)
    zq_t = z2d + (wsel_t - z2d)
    zq_ref[...] = jnp.swapaxes(zq_t.reshape(T, B, E_DIM), 0, 1)


def kernel(z, W):
    zq, loss, ind, v = pl.pallas_call(
        _vq_kernel,
        out_shape=[
            jax.ShapeDtypeStruct((B, T, E_DIM), jnp.float32),
            jax.ShapeDtypeStruct((1, 1), jnp.float32),
            jax.ShapeDtypeStruct((B, T), jnp.int32),
            jax.ShapeDtypeStruct((1, 1), jnp.int32),
        ],
    )(z, W)
    return (zq, loss.reshape(()), ind, v.reshape(()))
